# baseline (speedup 1.0000x reference)
import functools

import jax
import jax.numpy as jnp
from jax import lax
from jax.experimental import pallas as pl
from jax.experimental.pallas import tpu as pltpu
from jax.experimental.pallas import tpu_sc as plsc

B = 4096
L = 200
D = 64
H = 128
O = 10
HALF = L // 2
NVREG = D // 16


def _pool_sc(text2, emb):
    info = plsc.get_sparse_core_info()
    ncores = info.num_cores
    nw = ncores * info.num_subcores
    rpw = B // nw
    mesh = plsc.VectorSubcoreMesh(core_axis_name="c", subcore_axis_name="s")

    @functools.partial(
        pl.kernel,
        out_type=jax.ShapeDtypeStruct((B, D), jnp.float32),
        mesh=mesh,
        compiler_params=pltpu.CompilerParams(use_tc_tiling_on_sc=False),
        scratch_types=[
            pltpu.VMEM((2 * rpw, HALF), jnp.int32),
            pltpu.VMEM((L, D), jnp.float32),
            pltpu.VMEM((L, D), jnp.float32),
            pltpu.VMEM((rpw, D), jnp.float32),
            pltpu.SemaphoreType.DMA,
            pltpu.SemaphoreType.DMA,
        ],
    )
    def pool(text_hbm, emb_hbm, out_hbm, idx_v, rows0, rows1, out_v, sem0, sem1):
        wid = lax.axis_index("s") * ncores + lax.axis_index("c")
        base = wid * rpw
        pltpu.sync_copy(text_hbm.at[pl.ds(2 * base, 2 * rpw)], idx_v)
        bufs = (rows0, rows1)
        sems = (sem0, sem1)

        def issue(b, t):
            pltpu.async_copy(emb_hbm.at[idx_v.at[2 * b]],
                             bufs[t].at[pl.ds(0, HALF)], sems[t])
            pltpu.async_copy(emb_hbm.at[idx_v.at[2 * b + 1]],
                             bufs[t].at[pl.ds(HALF, HALF)], sems[t])

        def drain(t):
            pltpu.make_async_copy(emb_hbm.at[pl.ds(0, L)], bufs[t], sems[t]).wait()

        def consume(b, t):
            drain(t)
            buf = bufs[t]

            def rbody(r, acc):
                return tuple(acc[d] + buf[r, pl.ds(d * 16, 16)] for d in range(NVREG))

            acc = lax.fori_loop(0, L, rbody,
                                (jnp.zeros((16,), jnp.float32),) * NVREG,
                                unroll=8)
            inv = jnp.float32(1.0 / L)
            for d in range(NVREG):
                out_v[b, pl.ds(d * 16, 16)] = acc[d] * inv

        issue(0, 0)

        def outer(i, carry):
            for t in range(2):
                b = 2 * i + t

                @pl.when(b + 1 < rpw)
                def _():
                    issue(b + 1, (t + 1) % 2)

                consume(b, t)
            return carry

        lax.fori_loop(0, rpw // 2, outer, 0)
        pltpu.sync_copy(out_v, out_hbm.at[pl.ds(base, rpw)])

    return pool(text2, emb)


def _mlp_body(x_ref, w1_ref, b1_ref, w2_ref, b2_ref, o_ref):
    x = x_ref[...]
    h = lax.dot_general(x, w1_ref[...], (((1,), (1,)), ((), ())),
                        preferred_element_type=jnp.float32)
    h = jnp.maximum(h + b1_ref[...], 0.0)
    o = lax.dot_general(h, w2_ref[...], (((1,), (1,)), ((), ())),
                        preferred_element_type=jnp.float32)
    o_ref[...] = o + b2_ref[...]


def _mlp_tc(pooled, W1, b1, W2, b2):
    blk = 512
    return pl.pallas_call(
        _mlp_body,
        grid=(B // blk,),
        in_specs=[
            pl.BlockSpec((blk, D), lambda i: (i, 0)),
            pl.BlockSpec((H, D), lambda i: (0, 0)),
            pl.BlockSpec((1, H), lambda i: (0, 0)),
            pl.BlockSpec((O, H), lambda i: (0, 0)),
            pl.BlockSpec((1, O), lambda i: (0, 0)),
        ],
        out_specs=pl.BlockSpec((blk, O), lambda i: (i, 0)),
        out_shape=jax.ShapeDtypeStruct((B, O), jnp.float32),
    )(pooled, W1, b1.reshape(1, H), W2, b2.reshape(1, O))


def kernel(text, emb, W1, b1, W2, b2):
    text2 = text.astype(jnp.int32).reshape(2 * B, HALF)
    pooled = _pool_sc(text2, emb)
    return _mlp_tc(pooled, W1, b1, W2, b2)
